# baseline (speedup 1.0000x reference)
import functools

import jax
import jax.numpy as jnp
from jax import lax
from jax.experimental import pallas as pl
from jax.experimental.pallas import tpu as pltpu
from jax.experimental.pallas import tpu_sc as plsc

_NUM_TILES = 16
_CHUNK = 40


def _scatter_add_sc(E_n, idx4, n_nodes):
    b_dim, n_chunks, _, chunk = idx4.shape
    d = E_n.shape[-1]
    chunks_per_tile = n_chunks // _NUM_TILES
    node_stride = (n_nodes // _NUM_TILES) // 8 * 8
    node_size = n_nodes - (_NUM_TILES - 1) * node_stride

    mesh = plsc.VectorSubcoreMesh(core_axis_name="c", subcore_axis_name="s")

    nbuf = 8

    @functools.partial(
        pl.kernel,
        mesh=mesh,
        out_type=jax.ShapeDtypeStruct((b_dim, n_nodes, d), jnp.float32),
        scratch_types=[
            pltpu.VMEM((nbuf, chunk, d), jnp.float32),
            pltpu.VMEM((nbuf, 1, chunk), jnp.int32),
            pltpu.VMEM_SHARED((n_nodes, d), jnp.float32),
            pltpu.SemaphoreType.DMA,
            pltpu.SemaphoreType.DMA,
        ],
    )
    def scat(en_hbm, idx_hbm, out_hbm, eb, ib, agg_sh, gsem, ssem):
        b = lax.axis_index("c")
        s = lax.axis_index("s")
        base_node = s * node_stride
        base_chunk = s * chunks_per_tile
        base_edge = s * chunks_per_tile * chunk

        def gather(j, p):
            e_cp = pltpu.make_async_copy(
                en_hbm.at[b, pl.ds(base_edge + j * chunk, chunk)],
                eb.at[p], gsem)
            i_cp = pltpu.make_async_copy(
                idx_hbm.at[b, base_chunk + j], ib.at[p], gsem)
            return e_cp, i_cp

        def g_start(cps):
            cps[0].start()
            cps[1].start()

        def g_wait(cps):
            cps[0].wait()
            cps[1].wait()

        def scatter(p):
            return pltpu.make_async_copy(
                eb.at[p], agg_sh.at[ib.at[p, 0]], ssem)

        zeros16 = jnp.zeros((16,), jnp.float32)

        def zrow(r, carry):
            for k in range(d // 16):
                eb[0, r, pl.ds(k * 16, 16)] = zeros16
            return carry

        lax.fori_loop(0, chunk, zrow, 0)

        for i in range(node_size // chunk):
            pltpu.sync_copy(eb.at[0],
                            agg_sh.at[pl.ds(base_node + i * chunk, chunk)])

        plsc.subcore_barrier()

        glead, slag = nbuf - 2, 2

        def step(j, with_wait, with_start):
            p = j % nbuf
            g_wait(gather(j, p))
            scatter(p).start(add=True)
            if with_wait:
                scatter((j - slag) % nbuf).wait()
            if with_start:
                g_start(gather(j + glead, (j + glead) % nbuf))

        for j in range(glead):
            g_start(gather(j, j))
        for j in range(slag):
            step(j, False, True)

        def body(i, carry):
            j = nbuf * i + slag
            for k in range(nbuf):
                p = (slag + k) % nbuf
                jj = j + k
                g_wait(gather(jj, p))
                scatter(p).start(add=True)
                scatter((p - slag) % nbuf).wait()
                g_start(gather(jj + glead, (p + glead) % nbuf))
            return carry

        main_iters = (chunks_per_tile - slag - glead - 2) // nbuf
        lax.fori_loop(0, main_iters, body, 0)
        for j in range(slag + nbuf * main_iters, chunks_per_tile):
            step(j, True, j + glead < chunks_per_tile)
        for j in range(slag):
            scatter((chunks_per_tile - slag + j) % nbuf).wait()

        plsc.subcore_barrier()

        pltpu.sync_copy(agg_sh.at[pl.ds(base_node, node_size)],
                        out_hbm.at[b, pl.ds(base_node, node_size)])

    return scat


def _mlp_body(v_ref, a_ref, w1v_ref, w1a_ref, b1_ref, w2_ref, b2_ref,
              w3_ref, b3_ref, o_ref):
    h = jnp.dot(v_ref[...].astype(jnp.float32), w1v_ref[...],
                preferred_element_type=jnp.float32)
    h = h + jnp.dot(a_ref[...], w1a_ref[...], preferred_element_type=jnp.float32)
    h = jnp.maximum(h + b1_ref[...], 0.0)
    h = jnp.maximum(
        jnp.dot(h, w2_ref[...], preferred_element_type=jnp.float32) + b2_ref[...], 0.0)
    o_ref[...] = jnp.maximum(
        jnp.dot(h, w3_ref[...], preferred_element_type=jnp.float32) + b3_ref[...], 0.0)


def kernel(V, E_n, R_r, W1, b1, W2, b2, W3, b3):
    b_dim, n_nodes, dv = V.shape
    e_edges, de = E_n.shape[1], E_n.shape[2]
    dout = W1.shape[1]

    idx4 = R_r.reshape(b_dim, e_edges // _CHUNK, 1, _CHUNK)

    agg = _scatter_add_sc(E_n, idx4, n_nodes)(E_n, idx4)

    total = b_dim * n_nodes
    blk = 4000
    v2 = V.reshape(total, dv).astype(jnp.bfloat16)
    a2 = agg.reshape(total, de)
    w1v, w1a = W1[:dv], W1[dv:]

    h = pl.pallas_call(
        _mlp_body,
        grid=(total // blk,),
        in_specs=[
            pl.BlockSpec((blk, dv), lambda i: (i, 0)),
            pl.BlockSpec((blk, de), lambda i: (i, 0)),
            pl.BlockSpec((dv, dout), lambda i: (0, 0)),
            pl.BlockSpec((de, dout), lambda i: (0, 0)),
            pl.BlockSpec((1, dout), lambda i: (0, 0)),
            pl.BlockSpec((dout, dout), lambda i: (0, 0)),
            pl.BlockSpec((1, dout), lambda i: (0, 0)),
            pl.BlockSpec((dout, dout), lambda i: (0, 0)),
            pl.BlockSpec((1, dout), lambda i: (0, 0)),
        ],
        out_specs=pl.BlockSpec((blk, dout), lambda i: (i, 0)),
        out_shape=jax.ShapeDtypeStruct((total, dout), jnp.float32),
    )(v2, a2, w1v, w1a, b1[None, :], W2, b2[None, :], W3, b3[None, :])

    return h.reshape(b_dim, n_nodes, dout)


# speedup vs baseline: 1.0279x; 1.0279x over previous
import functools

import jax
import jax.numpy as jnp
from jax import lax
from jax.experimental import pallas as pl
from jax.experimental.pallas import tpu as pltpu
from jax.experimental.pallas import tpu_sc as plsc

_NUM_TILES = 16
_CHUNK = 80


def _scatter_add_sc(E_n, idx4, n_nodes):
    b_dim, n_chunks, _, chunk = idx4.shape
    d = E_n.shape[-1]
    chunks_per_tile = n_chunks // _NUM_TILES
    node_stride = (n_nodes // _NUM_TILES) // 8 * 8
    node_size = n_nodes - (_NUM_TILES - 1) * node_stride

    mesh = plsc.VectorSubcoreMesh(core_axis_name="c", subcore_axis_name="s")

    nbuf = 4

    @functools.partial(
        pl.kernel,
        mesh=mesh,
        out_type=jax.ShapeDtypeStruct((b_dim, n_nodes, d), jnp.float32),
        scratch_types=[
            pltpu.VMEM((nbuf, chunk, d), jnp.float32),
            pltpu.VMEM((nbuf, 1, chunk), jnp.int32),
            pltpu.VMEM_SHARED((n_nodes, d), jnp.float32),
            pltpu.SemaphoreType.DMA,
            pltpu.SemaphoreType.DMA,
        ],
    )
    def scat(en_hbm, idx_hbm, out_hbm, eb, ib, agg_sh, gsem, ssem):
        b = lax.axis_index("c")
        s = lax.axis_index("s")
        base_node = s * node_stride
        base_chunk = s * chunks_per_tile
        base_edge = s * chunks_per_tile * chunk

        def gather(j, p):
            e_cp = pltpu.make_async_copy(
                en_hbm.at[b, pl.ds(base_edge + j * chunk, chunk)],
                eb.at[p], gsem)
            i_cp = pltpu.make_async_copy(
                idx_hbm.at[b, base_chunk + j], ib.at[p], gsem)
            return e_cp, i_cp

        def g_start(cps):
            cps[0].start()
            cps[1].start()

        def g_wait(cps):
            cps[0].wait()
            cps[1].wait()

        def scatter(p):
            return pltpu.make_async_copy(
                eb.at[p], agg_sh.at[ib.at[p, 0]], ssem)

        zeros16 = jnp.zeros((16,), jnp.float32)

        def zrow(r, carry):
            for k in range(d // 16):
                eb[0, r, pl.ds(k * 16, 16)] = zeros16
            return carry

        lax.fori_loop(0, chunk, zrow, 0)

        for i in range(node_size // chunk):
            pltpu.sync_copy(eb.at[0],
                            agg_sh.at[pl.ds(base_node + i * chunk, chunk)])

        plsc.subcore_barrier()

        def step(j, with_wait, with_start):
            p = j % nbuf
            g_wait(gather(j, p))
            scatter(p).start(add=True)
            if with_wait:
                pw = (j - 1) % nbuf
                scatter(pw).wait()
            if with_start:
                g_start(gather(j + 3, (j + 3) % nbuf))

        g_start(gather(0, 0))
        g_start(gather(1, 1))
        g_start(gather(2, 2))
        step(0, False, True)
        for j in (1, 2, 3):
            step(j, True, True)

        def body(i, carry):
            j = nbuf * i + 4
            for k in range(nbuf):
                p = (4 + k) % nbuf
                jj = j + k
                g_wait(gather(jj, p))
                scatter(p).start(add=True)
                scatter((p - 1) % nbuf).wait()
                g_start(gather(jj + 3, (p + 3) % nbuf))
            return carry

        main_iters = (chunks_per_tile - 4 - 5) // nbuf
        lax.fori_loop(0, main_iters, body, 0)
        for j in range(4 + nbuf * main_iters, chunks_per_tile):
            step(j, True, j + 3 < chunks_per_tile)
        scatter((chunks_per_tile - 1) % nbuf).wait()

        plsc.subcore_barrier()

        pltpu.sync_copy(agg_sh.at[pl.ds(base_node, node_size)],
                        out_hbm.at[b, pl.ds(base_node, node_size)])

    return scat


def _mlp_body(v_ref, a_ref, w1v_ref, w1a_ref, b1_ref, w2_ref, b2_ref,
              w3_ref, b3_ref, o_ref):
    h = jnp.dot(v_ref[...].astype(jnp.float32), w1v_ref[...],
                preferred_element_type=jnp.float32)
    h = h + jnp.dot(a_ref[...], w1a_ref[...], preferred_element_type=jnp.float32)
    h = jnp.maximum(h + b1_ref[...], 0.0)
    h = jnp.maximum(
        jnp.dot(h, w2_ref[...], preferred_element_type=jnp.float32) + b2_ref[...], 0.0)
    o_ref[...] = jnp.maximum(
        jnp.dot(h, w3_ref[...], preferred_element_type=jnp.float32) + b3_ref[...], 0.0)


def kernel(V, E_n, R_r, W1, b1, W2, b2, W3, b3):
    b_dim, n_nodes, dv = V.shape
    e_edges, de = E_n.shape[1], E_n.shape[2]
    dout = W1.shape[1]

    idx4 = R_r.reshape(b_dim, e_edges // _CHUNK, 1, _CHUNK)

    agg = _scatter_add_sc(E_n, idx4, n_nodes)(E_n, idx4)

    total = b_dim * n_nodes
    blk = 4000
    v2 = V.reshape(total, dv).astype(jnp.bfloat16)
    a2 = agg.reshape(total, de)
    w1v, w1a = W1[:dv], W1[dv:]

    h = pl.pallas_call(
        _mlp_body,
        grid=(total // blk,),
        in_specs=[
            pl.BlockSpec((blk, dv), lambda i: (i, 0)),
            pl.BlockSpec((blk, de), lambda i: (i, 0)),
            pl.BlockSpec((dv, dout), lambda i: (0, 0)),
            pl.BlockSpec((de, dout), lambda i: (0, 0)),
            pl.BlockSpec((1, dout), lambda i: (0, 0)),
            pl.BlockSpec((dout, dout), lambda i: (0, 0)),
            pl.BlockSpec((1, dout), lambda i: (0, 0)),
            pl.BlockSpec((dout, dout), lambda i: (0, 0)),
            pl.BlockSpec((1, dout), lambda i: (0, 0)),
        ],
        out_specs=pl.BlockSpec((blk, dout), lambda i: (i, 0)),
        out_shape=jax.ShapeDtypeStruct((total, dout), jnp.float32),
    )(v2, a2, w1v, w1a, b1[None, :], W2, b2[None, :], W3, b3[None, :])

    return h.reshape(b_dim, n_nodes, dout)
